# baseline (speedup 1.0000x reference)
import functools

import jax
import jax.numpy as jnp
from jax import lax
from jax.experimental import pallas as pl
from jax.experimental.pallas import tpu as pltpu
from jax.experimental.pallas import tpu_sc as plsc

VOCAB = 1000000
DIM = 32
NVALS = 81920

_info = plsc.get_sparse_core_info()
_NC, _NS = _info.num_cores, _info.num_subcores
_NW = _NC * _NS
_BPW = NVALS // _NW


_mesh = plsc.VectorSubcoreMesh(core_axis_name="c", subcore_axis_name="s")


@functools.partial(
    pl.kernel,
    mesh=_mesh,
    out_type=jax.ShapeDtypeStruct((NVALS, DIM), jnp.float32),
    scratch_types=[
        pltpu.VMEM((_BPW,), jnp.int32),
        pltpu.VMEM((_BPW, DIM), jnp.float32),
        pltpu.SemaphoreType.DMA,
    ],
    compiler_params=pltpu.CompilerParams(use_tc_tiling_on_sc=False),
)
def _gather1(vals, tab, out, idx_v, rows_v, sem):
    wid = lax.axis_index("s") * _NC + lax.axis_index("c")
    base = wid * _BPW
    pltpu.sync_copy(vals.at[pl.ds(base, _BPW)], idx_v)
    pltpu.async_copy(tab.at[idx_v], rows_v, sem).wait()
    pltpu.sync_copy(rows_v, out.at[pl.ds(base, _BPW)])


def kernel(values_f1, lengths_f1, values_f2, lengths_f2,
           values_f3, lengths_f3, values_f4, lengths_f4,
           table_f1, table_f2, table_f3, table_f4):
    o1 = _gather1(values_f1, table_f1)
    o2 = _gather1(values_f2, table_f2)
    o3 = _gather1(values_f3, table_f3)
    o4 = _gather1(values_f4, table_f4)
    return (o1, lengths_f1, o2, lengths_f2, o3, lengths_f3, o4, lengths_f4)
